# Initial kernel scaffold; baseline (speedup 1.0000x reference)
import jax
import jax.numpy as jnp
from jax.experimental import pallas as pl


def kernel(x, emb1, emb2, W1, b1, W2, b2):
    raise NotImplementedError("write your pallas kernel here")


# speedup vs baseline: 5.5683x; 5.5683x over previous
import functools

import jax
import jax.numpy as jnp
from jax.experimental import pallas as pl

_ALPHA = 3.0
_K = 16


def _stage1_kernel(m1_ref, m2_ref, w1_ref, b1_ref, w2_ref, b2_ref, o1_ref, o2_ref):
    dn = (((1,), (1,)), ((), ()))
    p1 = jax.lax.dot_general(m1_ref[...], w1_ref[...], dn,
                             preferred_element_type=jnp.float32)
    o1_ref[...] = jnp.tanh(_ALPHA * (p1 + b1_ref[...]))
    p2 = jax.lax.dot_general(m2_ref[...], w2_ref[...], dn,
                             preferred_element_type=jnp.float32)
    o2_ref[...] = jnp.tanh(_ALPHA * (p2 + b2_ref[...]))


def _adj_kernel(m1_ref, m2_ref, noise_ref, out_ref, *, block_rows):
    i = pl.program_id(0)
    dn = (((1,), (1,)), ((), ()))
    m1b = m1_ref[pl.ds(i * block_rows, block_rows), :]
    m2b = m2_ref[pl.ds(i * block_rows, block_rows), :]
    raw = jax.lax.dot_general(m1b, m2_ref[...], dn,
                              preferred_element_type=jnp.float32)
    raw -= jax.lax.dot_general(m2b, m1_ref[...], dn,
                               preferred_element_type=jnp.float32)
    act = jnp.maximum(jnp.tanh(_ALPHA * raw), 0.0)
    an = act + noise_ref[...]
    m = jnp.max(an, axis=1, keepdims=True)
    for _ in range(_K - 1):
        m = jnp.max(jnp.where(an < m, an, -1.0), axis=1, keepdims=True)
    out_ref[...] = jnp.where(an >= m, act, 0.0)


def _pick_block_rows(n):
    for cand in (200, 80, 40, 16, 8):
        if n % cand == 0:
            return cand
    return n


def kernel(x, emb1, emb2, W1, b1, W2, b2):
    n = x.shape[0]
    dim = emb1.shape[1]
    m1 = jnp.take(emb1, x, axis=0)
    m2 = jnp.take(emb2, x, axis=0)
    M1, M2 = pl.pallas_call(
        _stage1_kernel,
        out_shape=(jax.ShapeDtypeStruct((n, dim), jnp.float32),
                   jax.ShapeDtypeStruct((n, dim), jnp.float32)),
    )(m1, m2, W1, b1.reshape(1, dim), W2, b2.reshape(1, dim))

    noise = jax.random.uniform(jax.random.key(1234), (n, n), jnp.float32) * 0.01

    br = _pick_block_rows(n)
    grid = n // br
    out = pl.pallas_call(
        functools.partial(_adj_kernel, block_rows=br),
        grid=(grid,),
        in_specs=[
            pl.BlockSpec((n, dim), lambda i: (0, 0)),
            pl.BlockSpec((n, dim), lambda i: (0, 0)),
            pl.BlockSpec((br, n), lambda i: (i, 0)),
        ],
        out_specs=pl.BlockSpec((br, n), lambda i: (i, 0)),
        out_shape=jax.ShapeDtypeStruct((n, n), jnp.float32),
    )(M1, M2, noise)
    return out
